# Initial kernel scaffold; baseline (speedup 1.0000x reference)
import jax
import jax.numpy as jnp
from jax.experimental import pallas as pl


def kernel(x, edge_index, edge_attr, h_prev, c_prev, Wn_i, We_i, b_i, Wn_f, We_f, b_f, Wn_c, We_c, b_c, Wn_o, We_o, b_o):
    raise NotImplementedError("write your pallas kernel here")


# speedup vs baseline: 1.8242x; 1.8242x over previous
import functools

import jax
import jax.numpy as jnp
from jax import lax
from jax.experimental import pallas as pl
from jax.experimental.pallas import tpu as pltpu
from jax.experimental.pallas import tpu_sc as plsc

N_NODES = 10000
N_EDGES = 320000
D_HID = 128
COMBINED = 256
D_EDGE = 16

NUM_CORES = 2
NUM_SUBCORES = 16
NUM_TILES = NUM_CORES * NUM_SUBCORES
EDGES_PER_TILE = N_EDGES // NUM_TILES
CHUNK = 80
NUM_CHUNKS = EDGES_PER_TILE // CHUNK
N_PAD = 10240
ROWS_PER_SUB = N_PAD // NUM_SUBCORES
ZROWS = 128


def _xs_body(a_ref, w_ref, oi_ref, of_ref, oc_ref, oo_ref):
    acc = jnp.dot(a_ref[...], w_ref[...], preferred_element_type=jnp.float32)
    oi_ref[...] = acc[:, 0:128]
    of_ref[...] = acc[:, 128:256]
    oc_ref[...] = acc[:, 256:384]
    oo_ref[...] = acc[:, 384:512]


def _xs_matmul(combined, Wn_all):
    rb = 1000
    grid = (N_NODES // rb,)
    out = jax.ShapeDtypeStruct((N_NODES, D_HID), jnp.float32)
    return pl.pallas_call(
        _xs_body,
        grid=grid,
        in_specs=[
            pl.BlockSpec((rb, COMBINED), lambda i: (i, 0)),
            pl.BlockSpec((COMBINED, 512), lambda i: (0, 0)),
        ],
        out_specs=[pl.BlockSpec((rb, D_HID), lambda i: (i, 0))] * 4,
        out_shape=[out] * 4,
    )(combined, Wn_all)


def _ep_body(a_ref, w_ref, b_ref, oi_ref, of_ref, oc_ref, oo_ref):
    acc = jnp.dot(a_ref[...], w_ref[...], preferred_element_type=jnp.float32)
    acc = acc + b_ref[...]
    oi_ref[...] = acc[:, 0:128]
    of_ref[...] = acc[:, 128:256]
    oc_ref[...] = acc[:, 256:384]
    oo_ref[...] = acc[:, 384:512]


def _ep_matmul(edge_attr, We_all, b_all):
    eb = 2000
    grid = (N_EDGES // eb,)
    out = jax.ShapeDtypeStruct((N_EDGES, D_HID), jnp.float32)
    return pl.pallas_call(
        _ep_body,
        grid=grid,
        in_specs=[
            pl.BlockSpec((eb, D_EDGE), lambda i: (i, 0)),
            pl.BlockSpec((D_EDGE, 512), lambda i: (0, 0)),
            pl.BlockSpec((1, 512), lambda i: (0, 0)),
        ],
        out_specs=[pl.BlockSpec((eb, D_HID), lambda i: (i, 0))] * 4,
        out_shape=[out] * 4,
    )(edge_attr, We_all, b_all)


def _sc_body(src_hbm, dst_hbm,
             xs_i, xs_f, xs_c, xs_o,
             ep_i, ep_f, ep_c, ep_o,
             out_i, out_f, out_c, out_o,
             sidx_v, didx_v, rows_v, ep_v, zbuf_v, agg_sh, sem):
    cid = lax.axis_index("c")
    sid = lax.axis_index("s")
    wid = cid * NUM_SUBCORES + sid
    ebase = wid * EDGES_PER_TILE

    def zrow(i, carry):
        for j in range(D_HID // 16):
            zbuf_v[i, pl.ds(j * 16, 16)] = jnp.zeros((16,), jnp.float32)
        return carry
    lax.fori_loop(0, ZROWS, zrow, 0)

    xs_refs = [xs_i, xs_f, xs_c, xs_o]
    ep_refs = [ep_i, ep_f, ep_c, ep_o]
    out_refs = [out_i, out_f, out_c, out_o]

    for g in range(4):
        for r in range(ROWS_PER_SUB // ZROWS):
            pltpu.sync_copy(zbuf_v,
                            agg_sh.at[pl.ds(sid * ROWS_PER_SUB + r * ZROWS, ZROWS)])
        plsc.subcore_barrier()

        def chunk(ci, carry):
            base = ebase + ci * CHUNK
            pltpu.sync_copy(src_hbm.at[pl.ds(base, CHUNK)], sidx_v)
            pltpu.sync_copy(dst_hbm.at[pl.ds(base, CHUNK)], didx_v)
            pltpu.async_copy(xs_refs[g].at[sidx_v], rows_v, sem).wait()
            pltpu.sync_copy(ep_refs[g].at[pl.ds(base, CHUNK)], ep_v)

            def row(i, c2):
                for j in range(D_HID // 16):
                    s = rows_v[i, pl.ds(j * 16, 16)] + ep_v[i, pl.ds(j * 16, 16)]
                    rows_v[i, pl.ds(j * 16, 16)] = jnp.maximum(s, 0.0)
                return c2
            lax.fori_loop(0, CHUNK, row, 0)

            pltpu.sync_copy(rows_v, agg_sh.at[didx_v], add=True)
            return carry
        lax.fori_loop(0, NUM_CHUNKS, chunk, 0)
        plsc.subcore_barrier()

        pltpu.sync_copy(agg_sh.at[pl.ds(sid * ROWS_PER_SUB, ROWS_PER_SUB)],
                        out_refs[g].at[cid, pl.ds(sid * ROWS_PER_SUB, ROWS_PER_SUB)])
        plsc.subcore_barrier()


def _sc_scatter(src, dst, xs, ep):
    part = jax.ShapeDtypeStruct((NUM_CORES, N_PAD, D_HID), jnp.float32)
    fn = pl.kernel(
        _sc_body,
        out_type=[part] * 4,
        mesh=plsc.VectorSubcoreMesh(core_axis_name="c", subcore_axis_name="s"),
        scratch_types=[
            pltpu.VMEM((CHUNK,), jnp.int32),
            pltpu.VMEM((CHUNK,), jnp.int32),
            pltpu.VMEM((CHUNK, D_HID), jnp.float32),
            pltpu.VMEM((CHUNK, D_HID), jnp.float32),
            pltpu.VMEM((ZROWS, D_HID), jnp.float32),
            pltpu.VMEM_SHARED((N_PAD, D_HID), jnp.float32),
            pltpu.SemaphoreType.DMA,
        ],
    )
    return fn(src, dst, *xs, *ep)


def _combine_body(pi_ref, pf_ref, pc_ref, po_ref, cprev_ref, h_ref, c_ref):
    gi = jax.nn.sigmoid(pi_ref[0] + pi_ref[1])
    gf = jax.nn.sigmoid(pf_ref[0] + pf_ref[1])
    gc = jnp.tanh(pc_ref[0] + pc_ref[1])
    go = jax.nn.sigmoid(po_ref[0] + po_ref[1])
    c = gf * cprev_ref[...] + gi * gc
    h_ref[...] = go * jnp.tanh(c)
    c_ref[...] = c


def _combine(parts, c_prev):
    rb = 1000
    grid = (N_NODES // rb,)
    out = jax.ShapeDtypeStruct((N_NODES, D_HID), jnp.float32)
    pspec = pl.BlockSpec((NUM_CORES, rb, D_HID), lambda i: (0, i, 0))
    return pl.pallas_call(
        _combine_body,
        grid=grid,
        in_specs=[pspec, pspec, pspec, pspec,
                  pl.BlockSpec((rb, D_HID), lambda i: (i, 0))],
        out_specs=[pl.BlockSpec((rb, D_HID), lambda i: (i, 0))] * 2,
        out_shape=[out, out],
    )(*parts, c_prev)


def kernel(x, edge_index, edge_attr, h_prev, c_prev,
           Wn_i, We_i, b_i, Wn_f, We_f, b_f,
           Wn_c, We_c, b_c, Wn_o, We_o, b_o):
    src = edge_index[0].astype(jnp.int32)
    dst = edge_index[1].astype(jnp.int32)
    combined = jnp.concatenate([x, h_prev], axis=1)
    Wn_all = jnp.concatenate([Wn_i, Wn_f, Wn_c, Wn_o], axis=1)
    We_all = jnp.concatenate([We_i, We_f, We_c, We_o], axis=1)
    b_all = jnp.concatenate([b_i, b_f, b_c, b_o]).reshape(1, 512)

    xs = _xs_matmul(combined, Wn_all)
    ep = _ep_matmul(edge_attr, We_all, b_all)
    parts = _sc_scatter(src, dst, xs, ep)
    h, c = _combine(parts, c_prev)
    return (h, c)
